# Initial kernel scaffold; baseline (speedup 1.0000x reference)
import jax
import jax.numpy as jnp
from jax.experimental import pallas as pl


def kernel(feature_shape, all_rois_center, rois_feature_usps):
    raise NotImplementedError("write your pallas kernel here")


# speedup vs baseline: 1.7737x; 1.7737x over previous
import functools

import jax
import jax.numpy as jnp
from jax import lax
from jax.experimental import pallas as pl
from jax.experimental.pallas import tpu as pltpu

LEVELS = 4
N = 4
C = 128
H = 128
W = 128
R = 6272
RB = 784
J = R // RB


def _weights_body(x_ref, y_ref, f_ref, rows_ref, idx_ref):
    x = x_ref[...]
    y = y_ref[...]
    f = f_ref[...] * 0.25
    fx = jnp.floor(x)
    fy = jnp.floor(y)
    xp = x - fx
    yp = y - fy
    vx = xp * xp + (1.0 - xp) * (1.0 - xp)
    vy = yp * yp + (1.0 - yp) * (1.0 - yp)
    invq = 1.0 / (vx * vy)
    fxi = fx.astype(jnp.int32)
    fyi = fy.astype(jnp.int32)
    cxi = fxi + 1
    cyi = fyi + 1
    u0 = (1.0 - xp) * invq
    u1 = xp * invq
    v0 = 1.0 - yp
    v1 = yp
    bx0 = fxi < H
    bx1 = cxi < H
    by0 = fyi < H
    by1 = cyi < H
    zf = jnp.zeros_like(x)
    zi = jnp.zeros_like(fxi)
    m11 = bx0 & by0
    m12 = bx0 & by1
    m21 = bx1 & by0
    m22 = bx1 & by1
    rows_ref[0, :, :] = jnp.where(m11, u0 * v0, zf) * f
    rows_ref[1, :, :] = jnp.where(m12, u0 * v1, zf) * f
    rows_ref[2, :, :] = jnp.where(m21, u1 * v0, zf) * f
    rows_ref[3, :, :] = jnp.where(m22, u1 * v1, zf) * f
    idx_ref[0, :, :] = jnp.where(m11, fxi * W + fyi, zi)
    idx_ref[1, :, :] = jnp.where(m12, fxi * W + cyi, zi)
    idx_ref[2, :, :] = jnp.where(m21, cxi * W + fyi, zi)
    idx_ref[3, :, :] = jnp.where(m22, cxi * W + cyi, zi)


def _phase_a(arc, fr):
    return pl.pallas_call(
        _weights_body,
        grid=(N, LEVELS, J),
        in_specs=[
            pl.BlockSpec((None, None, RB, 1),
                         lambda n, l, j: (l, n, j, 0)),
            pl.BlockSpec((None, None, RB, 1),
                         lambda n, l, j: (l, n, j, 0)),
            pl.BlockSpec((None, None, RB, C),
                         lambda n, l, j: (l, n, j, 0)),
        ],
        out_specs=[
            pl.BlockSpec((None, None, 4, RB, C),
                         lambda n, l, j: (n, l, 0, j, 0)),
            pl.BlockSpec((None, None, 4, RB, 1),
                         lambda n, l, j: (n, l, 0, j, 0)),
        ],
        out_shape=[
            jax.ShapeDtypeStruct((N, LEVELS, 4, R, C), jnp.float32),
            jax.ShapeDtypeStruct((N, LEVELS, 4, R, 1), jnp.int32),
        ],
    )(arc[:, 0], arc[:, 1], fr)


def _transpose_body(in_ref, out_ref):
    out_ref[...] = in_ref[...].T


def _phase_c(acc):
    HW = H * W
    BLK = 1024
    return pl.pallas_call(
        _transpose_body,
        grid=(N, HW // BLK),
        in_specs=[pl.BlockSpec((None, BLK, C), lambda n, j: (n, j, 0))],
        out_specs=pl.BlockSpec((None, C, BLK), lambda n, j: (n, 0, j)),
        out_shape=jax.ShapeDtypeStruct((N, C, HW), jnp.float32),
    )(acc)


def kernel(feature_shape, all_rois_center, rois_feature_usps):
    arc = all_rois_center.reshape(LEVELS, 2, N, R, 1)
    fr = rois_feature_usps.reshape(LEVELS, N, R, C)
    rows, idx = _phase_a(arc, fr)
    rows2 = rows.reshape(N, LEVELS * 4 * R, C)
    idx2 = idx.reshape(N, LEVELS * 4 * R)
    acc = jax.vmap(
        lambda r, i: jnp.zeros((H * W, C), jnp.float32).at[i].add(r)
    )(rows2, idx2)
    out = _phase_c(acc)
    return out.reshape(N, C, H, W)


# speedup vs baseline: 4.2171x; 2.3776x over previous
import functools

import jax
import jax.numpy as jnp
from jax import lax
from jax.experimental import pallas as pl
from jax.experimental.pallas import tpu as pltpu
from jax.experimental.pallas import tpu_sc as plsc

LEVELS = 4
N = 4
C = 128
H = 128
W = 128
R = 6272
RB = 784
J = R // RB

TOT = LEVELS * 4 * R
NSUB = 16
NCORE = 2
TPT = TOT // NSUB
CH = 128
NCK = TPT // CH
CHALF = C // NCORE
HW = H * W
RPT = HW // NSUB


def _weights_body(x_ref, y_ref, f_ref, rows_ref, idx_ref):
    x = x_ref[...]
    y = y_ref[...]
    f = f_ref[...] * 0.25
    fx = jnp.floor(x)
    fy = jnp.floor(y)
    xp = x - fx
    yp = y - fy
    vx = xp * xp + (1.0 - xp) * (1.0 - xp)
    vy = yp * yp + (1.0 - yp) * (1.0 - yp)
    invq = 1.0 / (vx * vy)
    fxi = fx.astype(jnp.int32)
    fyi = fy.astype(jnp.int32)
    cxi = fxi + 1
    cyi = fyi + 1
    u0 = (1.0 - xp) * invq
    u1 = xp * invq
    v0 = 1.0 - yp
    v1 = yp
    bx0 = fxi < H
    bx1 = cxi < H
    by0 = fyi < H
    by1 = cyi < H
    zf = jnp.zeros_like(x)
    zi = jnp.zeros_like(fxi)
    m11 = bx0 & by0
    m12 = bx0 & by1
    m21 = bx1 & by0
    m22 = bx1 & by1
    rows_ref[0, :, :] = jnp.where(m11, u0 * v0, zf) * f
    rows_ref[1, :, :] = jnp.where(m12, u0 * v1, zf) * f
    rows_ref[2, :, :] = jnp.where(m21, u1 * v0, zf) * f
    rows_ref[3, :, :] = jnp.where(m22, u1 * v1, zf) * f
    idx_ref[0, :, :] = jnp.where(m11, fxi * W + fyi, zi)
    idx_ref[1, :, :] = jnp.where(m12, fxi * W + cyi, zi)
    idx_ref[2, :, :] = jnp.where(m21, cxi * W + fyi, zi)
    idx_ref[3, :, :] = jnp.where(m22, cxi * W + cyi, zi)


def _phase_a(arc, fr):
    return pl.pallas_call(
        _weights_body,
        grid=(N, LEVELS, J),
        in_specs=[
            pl.BlockSpec((None, None, RB, 1),
                         lambda n, l, j: (l, n, j, 0)),
            pl.BlockSpec((None, None, RB, 1),
                         lambda n, l, j: (l, n, j, 0)),
            pl.BlockSpec((None, None, RB, C),
                         lambda n, l, j: (l, n, j, 0)),
        ],
        out_specs=[
            pl.BlockSpec((None, None, 4, RB, C),
                         lambda n, l, j: (n, l, 0, j, 0)),
            pl.BlockSpec((None, None, 4, RB, 1),
                         lambda n, l, j: (n, l, 0, j, 0)),
        ],
        out_shape=[
            jax.ShapeDtypeStruct((N, LEVELS, 4, R, C), jnp.float32),
            jax.ShapeDtypeStruct((N, LEVELS, 4, R, 1), jnp.int32),
        ],
    )(arc[:, 0], arc[:, 1], fr)


def _sc_body(rows_hbm, idx_hbm, zeros_hbm, out_hbm, idx_v, rows_v, acc, sem):
    cid = lax.axis_index("c")
    sid = lax.axis_index("s")
    col0 = cid * CHALF
    for n in range(N):
        pltpu.sync_copy(zeros_hbm, acc.at[pl.ds(sid * RPT, RPT)])
        pltpu.sync_copy(idx_hbm.at[n, sid], idx_v)
        plsc.subcore_barrier()
        base = sid * TPT

        def chunk_body(k, _):
            pltpu.sync_copy(
                rows_hbm.at[n, pl.ds(base + k * CH, CH), pl.ds(col0, CHALF)],
                rows_v)
            pltpu.sync_copy(rows_v, acc.at[idx_v.at[k]], add=True)
            return _

        lax.fori_loop(0, NCK, chunk_body, None)
        plsc.subcore_barrier()
        pltpu.sync_copy(
            acc.at[pl.ds(sid * RPT, RPT)],
            out_hbm.at[n, pl.ds(sid * RPT, RPT), pl.ds(col0, CHALF)])
        plsc.subcore_barrier()


def _phase_b(rows2, idx3, zeros):
    mesh = plsc.VectorSubcoreMesh(core_axis_name="c", subcore_axis_name="s")
    f = pl.kernel(
        _sc_body,
        out_type=jax.ShapeDtypeStruct((N, HW, C), jnp.float32),
        mesh=mesh,
        scratch_types=[
            pltpu.VMEM((NCK, CH), jnp.int32),
            pltpu.VMEM((CH, CHALF), jnp.float32),
            pltpu.VMEM_SHARED((HW, CHALF), jnp.float32),
            pltpu.SemaphoreType.DMA,
        ],
        compiler_params=pltpu.CompilerParams(use_tc_tiling_on_sc=False),
    )
    return f(rows2, idx3, zeros)


def _transpose_body(in_ref, out_ref):
    out_ref[...] = in_ref[...].T


def _phase_c(acc):
    HW = H * W
    BLK = 1024
    return pl.pallas_call(
        _transpose_body,
        grid=(N, HW // BLK),
        in_specs=[pl.BlockSpec((None, BLK, C), lambda n, j: (n, j, 0))],
        out_specs=pl.BlockSpec((None, C, BLK), lambda n, j: (n, 0, j)),
        out_shape=jax.ShapeDtypeStruct((N, C, HW), jnp.float32),
    )(acc)


def kernel(feature_shape, all_rois_center, rois_feature_usps):
    arc = all_rois_center.reshape(LEVELS, 2, N, R, 1)
    fr = rois_feature_usps.reshape(LEVELS, N, R, C)
    rows, idx = _phase_a(arc, fr)
    rows2 = rows.reshape(N, TOT, C)
    idx3 = idx.reshape(N, NSUB, NCK, CH)
    zeros = jnp.zeros((RPT, CHALF), jnp.float32)
    acc = _phase_b(rows2, idx3, zeros)
    out = _phase_c(acc)
    return out.reshape(N, C, H, W)


# speedup vs baseline: 4.9846x; 1.1820x over previous
import functools

import jax
import jax.numpy as jnp
from jax import lax
from jax.experimental import pallas as pl
from jax.experimental.pallas import tpu as pltpu
from jax.experimental.pallas import tpu_sc as plsc

LEVELS = 4
N = 4
C = 128
H = 128
W = 128
R = 6272
RB = 784
J = R // RB

TOT = LEVELS * 4 * R
NSUB = 16
NCORE = 2
TPT = TOT // NSUB
CH = 128
NCK = TPT // CH
CHALF = C // NCORE
HW = H * W
RPT = HW // NSUB


def _weights_body(x_ref, y_ref, f_ref, rows_ref, idx_ref):
    x = x_ref[...]
    y = y_ref[...]
    f = f_ref[...] * 0.25
    fx = jnp.floor(x)
    fy = jnp.floor(y)
    xp = x - fx
    yp = y - fy
    vx = xp * xp + (1.0 - xp) * (1.0 - xp)
    vy = yp * yp + (1.0 - yp) * (1.0 - yp)
    invq = 1.0 / (vx * vy)
    fxi = fx.astype(jnp.int32)
    fyi = fy.astype(jnp.int32)
    cxi = fxi + 1
    cyi = fyi + 1
    u0 = (1.0 - xp) * invq
    u1 = xp * invq
    v0 = 1.0 - yp
    v1 = yp
    bx0 = fxi < H
    bx1 = cxi < H
    by0 = fyi < H
    by1 = cyi < H
    zf = jnp.zeros_like(x)
    zi = jnp.zeros_like(fxi)
    m11 = bx0 & by0
    m12 = bx0 & by1
    m21 = bx1 & by0
    m22 = bx1 & by1
    rows_ref[0, :, :] = jnp.where(m11, u0 * v0, zf) * f
    rows_ref[1, :, :] = jnp.where(m12, u0 * v1, zf) * f
    rows_ref[2, :, :] = jnp.where(m21, u1 * v0, zf) * f
    rows_ref[3, :, :] = jnp.where(m22, u1 * v1, zf) * f
    idx_ref[0, :, :] = jnp.where(m11, fxi * W + fyi, zi)
    idx_ref[1, :, :] = jnp.where(m12, fxi * W + cyi, zi)
    idx_ref[2, :, :] = jnp.where(m21, cxi * W + fyi, zi)
    idx_ref[3, :, :] = jnp.where(m22, cxi * W + cyi, zi)


def _phase_a(arc, fr):
    return pl.pallas_call(
        _weights_body,
        grid=(N, LEVELS, J),
        in_specs=[
            pl.BlockSpec((None, None, RB, 1),
                         lambda n, l, j: (l, n, j, 0)),
            pl.BlockSpec((None, None, RB, 1),
                         lambda n, l, j: (l, n, j, 0)),
            pl.BlockSpec((None, None, RB, C),
                         lambda n, l, j: (l, n, j, 0)),
        ],
        out_specs=[
            pl.BlockSpec((None, None, 4, RB, C),
                         lambda n, l, j: (n, l, 0, j, 0)),
            pl.BlockSpec((None, None, 4, RB, 1),
                         lambda n, l, j: (n, l, 0, j, 0)),
        ],
        out_shape=[
            jax.ShapeDtypeStruct((N, LEVELS, 4, R, C), jnp.float32),
            jax.ShapeDtypeStruct((N, LEVELS, 4, R, 1), jnp.int32),
        ],
    )(arc[:, 0], arc[:, 1], fr)


def _sc_body(rows_hbm, idx_hbm, zeros_hbm, out_hbm, idx_v, rows_v0, rows_v1,
             acc, sem0, sem1):
    cid = lax.axis_index("c")
    sid = lax.axis_index("s")
    col0 = cid * CHALF
    slots = ((rows_v0, sem0), (rows_v1, sem1))

    for n in range(N):
        pltpu.sync_copy(zeros_hbm, acc.at[pl.ds(sid * RPT, RPT)])
        pltpu.sync_copy(idx_hbm.at[n, sid], idx_v)
        plsc.subcore_barrier()
        base = sid * TPT

        def start(k, slot):
            buf, sem = slots[slot]
            return pltpu.async_copy(
                rows_hbm.at[n, pl.ds(base + k * CH, CH), pl.ds(col0, CHALF)],
                buf, sem)

        def scatter(k, slot):
            buf, sem = slots[slot]
            pltpu.make_async_copy(
                rows_hbm.at[n, pl.ds(base, CH), pl.ds(col0, CHALF)],
                buf, sem).wait()
            pltpu.sync_copy(buf, acc.at[idx_v.at[k]], add=True)

        start(0, 0)
        start(1, 1)

        def pair_body(i, _):
            k0 = 2 * i
            scatter(k0, 0)
            start(jnp.minimum(k0 + 2, NCK - 1), 0)
            scatter(k0 + 1, 1)
            start(jnp.minimum(k0 + 3, NCK - 1), 1)
            return _

        lax.fori_loop(0, (NCK - 1) // 2, pair_body, None)
        scatter(NCK - 1, 0)
        pltpu.make_async_copy(
            rows_hbm.at[n, pl.ds(base, CH), pl.ds(col0, CHALF)],
            rows_v1, sem1).wait()
        plsc.subcore_barrier()
        pltpu.sync_copy(
            acc.at[pl.ds(sid * RPT, RPT)],
            out_hbm.at[n, pl.ds(sid * RPT, RPT), pl.ds(col0, CHALF)])
        plsc.subcore_barrier()


def _phase_b(rows2, idx3, zeros):
    mesh = plsc.VectorSubcoreMesh(core_axis_name="c", subcore_axis_name="s")
    f = pl.kernel(
        _sc_body,
        out_type=jax.ShapeDtypeStruct((N, HW, C), jnp.float32),
        mesh=mesh,
        scratch_types=[
            pltpu.VMEM((NCK, CH), jnp.int32),
            pltpu.VMEM((CH, CHALF), jnp.float32),
            pltpu.VMEM((CH, CHALF), jnp.float32),
            pltpu.VMEM_SHARED((HW, CHALF), jnp.float32),
            pltpu.SemaphoreType.DMA,
            pltpu.SemaphoreType.DMA,
        ],
        compiler_params=pltpu.CompilerParams(use_tc_tiling_on_sc=False),
    )
    return f(rows2, idx3, zeros)


def _transpose_body(in_ref, out_ref):
    out_ref[...] = in_ref[...].T


def _phase_c(acc):
    HW = H * W
    BLK = 1024
    return pl.pallas_call(
        _transpose_body,
        grid=(N, HW // BLK),
        in_specs=[pl.BlockSpec((None, BLK, C), lambda n, j: (n, j, 0))],
        out_specs=pl.BlockSpec((None, C, BLK), lambda n, j: (n, 0, j)),
        out_shape=jax.ShapeDtypeStruct((N, C, HW), jnp.float32),
    )(acc)


def kernel(feature_shape, all_rois_center, rois_feature_usps):
    arc = all_rois_center.reshape(LEVELS, 2, N, R, 1)
    fr = rois_feature_usps.reshape(LEVELS, N, R, C)
    rows, idx = _phase_a(arc, fr)
    rows2 = rows.reshape(N, TOT, C)
    idx3 = idx.reshape(N, NSUB, NCK, CH)
    zeros = jnp.zeros((RPT, CHALF), jnp.float32)
    acc = _phase_b(rows2, idx3, zeros)
    out = _phase_c(acc)
    return out.reshape(N, C, H, W)


# speedup vs baseline: 5.0570x; 1.0145x over previous
import functools

import jax
import jax.numpy as jnp
from jax import lax
from jax.experimental import pallas as pl
from jax.experimental.pallas import tpu as pltpu
from jax.experimental.pallas import tpu_sc as plsc

LEVELS = 4
N = 4
C = 128
H = 128
W = 128
R = 6272
RB = 784
J = R // RB

TOT = LEVELS * 4 * R
NSUB = 16
NCORE = 2
TPT = TOT // NSUB
CH = 128
NCK = TPT // CH
CHALF = C // NCORE
HW = H * W
RPT = HW // NSUB


def _weights_body(x_ref, y_ref, f_ref, rows_ref, idx_ref):
    x = x_ref[...]
    y = y_ref[...]
    f = f_ref[...] * 0.25
    fx = jnp.floor(x)
    fy = jnp.floor(y)
    xp = x - fx
    yp = y - fy
    vx = xp * xp + (1.0 - xp) * (1.0 - xp)
    vy = yp * yp + (1.0 - yp) * (1.0 - yp)
    invq = 1.0 / (vx * vy)
    fxi = fx.astype(jnp.int32)
    fyi = fy.astype(jnp.int32)
    cxi = fxi + 1
    cyi = fyi + 1
    u0 = (1.0 - xp) * invq
    u1 = xp * invq
    v0 = 1.0 - yp
    v1 = yp
    bx0 = fxi < H
    bx1 = cxi < H
    by0 = fyi < H
    by1 = cyi < H
    zf = jnp.zeros_like(x)
    zi = jnp.zeros_like(fxi)
    m11 = bx0 & by0
    m12 = bx0 & by1
    m21 = bx1 & by0
    m22 = bx1 & by1
    rows_ref[0, :, :] = jnp.where(m11, u0 * v0, zf) * f
    rows_ref[1, :, :] = jnp.where(m12, u0 * v1, zf) * f
    rows_ref[2, :, :] = jnp.where(m21, u1 * v0, zf) * f
    rows_ref[3, :, :] = jnp.where(m22, u1 * v1, zf) * f
    idx_ref[0, :, :] = jnp.where(m11, fxi * W + fyi, zi)
    idx_ref[1, :, :] = jnp.where(m12, fxi * W + cyi, zi)
    idx_ref[2, :, :] = jnp.where(m21, cxi * W + fyi, zi)
    idx_ref[3, :, :] = jnp.where(m22, cxi * W + cyi, zi)


def _phase_a(arc, fr):
    return pl.pallas_call(
        _weights_body,
        grid=(N, LEVELS, J),
        in_specs=[
            pl.BlockSpec((None, None, RB, 1),
                         lambda n, l, j: (l, n, j, 0)),
            pl.BlockSpec((None, None, RB, 1),
                         lambda n, l, j: (l, n, j, 0)),
            pl.BlockSpec((None, None, RB, C),
                         lambda n, l, j: (l, n, j, 0)),
        ],
        out_specs=[
            pl.BlockSpec((None, None, 4, RB, C),
                         lambda n, l, j: (n, l, 0, j, 0)),
            pl.BlockSpec((None, None, 4, RB, 1),
                         lambda n, l, j: (n, l, 0, j, 0)),
        ],
        out_shape=[
            jax.ShapeDtypeStruct((N, LEVELS, 4, R, C), jnp.float32),
            jax.ShapeDtypeStruct((N, LEVELS, 4, R, 1), jnp.int32),
        ],
    )(arc[:, 0], arc[:, 1], fr)


NB = 4


def _sc_body(rows_hbm, idx_hbm, zeros_hbm, out_hbm, idx_v, bufs, gsems, ssems,
             acc):
    cid = lax.axis_index("c")
    sid = lax.axis_index("s")
    col0 = cid * CHALF

    for n in range(N):
        pltpu.sync_copy(zeros_hbm, acc.at[pl.ds(sid * RPT, RPT)])
        pltpu.sync_copy(idx_hbm.at[n, sid], idx_v)
        plsc.subcore_barrier()
        base = sid * TPT

        def gather_start(k, s):
            pltpu.async_copy(
                rows_hbm.at[n, pl.ds(base + k * CH, CH), pl.ds(col0, CHALF)],
                bufs[s], gsems[s])

        def gather_wait(s):
            pltpu.make_async_copy(
                rows_hbm.at[n, pl.ds(base, CH), pl.ds(col0, CHALF)],
                bufs[s], gsems[s]).wait()

        def scat_start(k, s):
            pltpu.async_copy(bufs[s], acc.at[idx_v.at[k]], ssems[s], add=True)

        def scat_wait(k, s):
            pltpu.make_async_copy(
                bufs[s], acc.at[idx_v.at[k]], ssems[s]).wait()

        gather_start(0, 0)
        gather_start(1, 1)
        gather_wait(0)
        scat_start(0, 0)
        gather_start(2, 2)
        gather_wait(1)
        scat_start(1, 1)
        gather_start(3, 3)

        def quad_body(i, _):
            k0 = NB * i + 2
            for s in range(NB):
                k = k0 + s
                sl = (2 + s) % NB
                gather_wait(sl)
                scat_start(k, sl)
                scat_wait(k - 2, (sl + 2) % NB)
                gather_start(k + 2, (sl + 2) % NB)
            return _

        lax.fori_loop(0, 11, quad_body, None)
        k46 = jnp.int32(46)
        gather_wait(2)
        scat_start(k46, 2)
        scat_wait(k46 - 2, 0)
        gather_start(k46 + 2, 0)
        gather_wait(3)
        scat_start(k46 + 1, 3)
        scat_wait(k46 - 1, 1)
        gather_wait(0)
        scat_start(k46 + 2, 0)
        scat_wait(k46, 2)
        scat_wait(k46 + 1, 3)
        scat_wait(k46 + 2, 0)
        plsc.subcore_barrier()
        pltpu.sync_copy(
            acc.at[pl.ds(sid * RPT, RPT)],
            out_hbm.at[n, pl.ds(sid * RPT, RPT), pl.ds(col0, CHALF)])
        plsc.subcore_barrier()


def _phase_b(rows2, idx3, zeros):
    mesh = plsc.VectorSubcoreMesh(core_axis_name="c", subcore_axis_name="s")
    f = pl.kernel(
        _sc_body,
        out_type=jax.ShapeDtypeStruct((N, HW, C), jnp.float32),
        mesh=mesh,
        scratch_types=[
            pltpu.VMEM((NCK, CH), jnp.int32),
            tuple(pltpu.VMEM((CH, CHALF), jnp.float32) for _ in range(NB)),
            tuple(pltpu.SemaphoreType.DMA for _ in range(NB)),
            tuple(pltpu.SemaphoreType.DMA for _ in range(NB)),
            pltpu.VMEM_SHARED((HW, CHALF), jnp.float32),
        ],
        compiler_params=pltpu.CompilerParams(use_tc_tiling_on_sc=False),
    )
    return f(rows2, idx3, zeros)


def _transpose_body(in_ref, out_ref):
    out_ref[...] = in_ref[...].T


def _phase_c(acc):
    HW = H * W
    BLK = 1024
    return pl.pallas_call(
        _transpose_body,
        grid=(N, HW // BLK),
        in_specs=[pl.BlockSpec((None, BLK, C), lambda n, j: (n, j, 0))],
        out_specs=pl.BlockSpec((None, C, BLK), lambda n, j: (n, 0, j)),
        out_shape=jax.ShapeDtypeStruct((N, C, HW), jnp.float32),
    )(acc)


def kernel(feature_shape, all_rois_center, rois_feature_usps):
    arc = all_rois_center.reshape(LEVELS, 2, N, R, 1)
    fr = rois_feature_usps.reshape(LEVELS, N, R, C)
    rows, idx = _phase_a(arc, fr)
    rows2 = rows.reshape(N, TOT, C)
    idx3 = idx.reshape(N, NSUB, NCK, CH)
    zeros = jnp.zeros((RPT, CHALF), jnp.float32)
    acc = _phase_b(rows2, idx3, zeros)
    out = _phase_c(acc)
    return out.reshape(N, C, H, W)
